# Initial kernel scaffold; baseline (speedup 1.0000x reference)
import jax
import jax.numpy as jnp
from jax.experimental import pallas as pl


def kernel(x, edge_index, eps, W1, b1, W2, b2, W3, b3, W4, b4, W_mu, b_mu, W_ls, b_ls):
    raise NotImplementedError("write your pallas kernel here")


# speedup vs baseline: 11.8550x; 11.8550x over previous
import functools

import jax
import jax.numpy as jnp
from jax import lax
from jax.experimental import pallas as pl
from jax.experimental.pallas import tpu as pltpu
from jax.experimental.pallas import tpu_sc as plsc

N_NODES = 10000
N_EDGES = 320000
N_PAD = 10240
NW = 32
NSC = 16
K = 128
CH = 79
E_PAD = NW * CH * K
RPS = N_PAD // NSC
BR = 1280

_MESH = plsc.VectorSubcoreMesh(core_axis_name="c", subcore_axis_name="s")
_PREC = jax.lax.Precision.HIGHEST
_SC_PARAMS = pltpu.CompilerParams(use_tc_tiling_on_sc=False)


def _zero_fill(stage, f):
    zeros = jnp.zeros((16,), jnp.float32)

    @pl.loop(0, K)
    def _(r):
        for cc in range(f // 16):
            stage[r, pl.ds(cc * 16, 16)] = zeros


def _stripe_zero(stage, acc, s):
    for b in range(RPS // K):
        pltpu.sync_copy(stage, acc.at[pl.ds(s * RPS + b * K, K)])


@functools.partial(
    pl.kernel,
    out_type=jax.ShapeDtypeStruct((2, N_PAD, 16), jnp.float32),
    mesh=_MESH,
    scratch_types=[
        pltpu.VMEM((CH, K), jnp.int32),
        pltpu.VMEM((K, 16), jnp.float32),
        pltpu.VMEM_SHARED((N_PAD, 16), jnp.float32),
    ],
    compiler_params=_SC_PARAMS,
)
def _sc_degree(dst_hbm, out_hbm, dst_v, stage, acc):
    c = lax.axis_index("c")
    s = lax.axis_index("s")
    wid = c * NSC + s
    pltpu.sync_copy(dst_hbm.at[wid], dst_v)
    _zero_fill(stage, 16)
    _stripe_zero(stage, acc, s)
    basis = jnp.where(lax.iota(jnp.int32, 16) == 0, 1.0, 0.0)

    @pl.loop(0, K)
    def _(r):
        stage[r, pl.ds(0, 16)] = basis

    plsc.subcore_barrier()

    @pl.loop(0, CH)
    def _(j):
        pltpu.sync_copy(stage, acc.at[dst_v.at[j]], add=True)

    plsc.subcore_barrier()
    pltpu.sync_copy(acc.at[pl.ds(s * RPS, RPS)],
                    out_hbm.at[c, pl.ds(s * RPS, RPS)])


@functools.lru_cache(maxsize=None)
def _make_prop(f):
    @functools.partial(
        pl.kernel,
        out_type=jax.ShapeDtypeStruct((2, N_PAD, f), jnp.float32),
        mesh=_MESH,
        scratch_types=[
            pltpu.VMEM((CH, K), jnp.int32),
            pltpu.VMEM((CH, K), jnp.int32),
            pltpu.VMEM((K, f), jnp.float32),
            pltpu.VMEM_SHARED((N_PAD, f), jnp.float32),
            pltpu.SemaphoreType.DMA,
        ],
        compiler_params=_SC_PARAMS,
    )
    def prop(u_hbm, src_hbm, dst_hbm, out_hbm, src_v, dst_v, stage, acc, sem):
        c = lax.axis_index("c")
        s = lax.axis_index("s")
        wid = c * NSC + s
        pltpu.sync_copy(src_hbm.at[wid], src_v)
        pltpu.sync_copy(dst_hbm.at[wid], dst_v)
        _zero_fill(stage, f)
        _stripe_zero(stage, acc, s)
        plsc.subcore_barrier()

        @pl.loop(0, CH)
        def _(j):
            pltpu.async_copy(u_hbm.at[src_v.at[j]], stage, sem).wait()
            pltpu.sync_copy(stage, acc.at[dst_v.at[j]], add=True)

        plsc.subcore_barrier()
        pltpu.sync_copy(acc.at[pl.ds(s * RPS, RPS)],
                        out_hbm.at[c, pl.ds(s * RPS, RPS)])

    return prop


def _row_specs(*widths):
    return [pl.BlockSpec((BR, w), lambda i: (i, 0)) for w in widths]


def _full_spec(shape):
    return pl.BlockSpec(shape, lambda i: (0, 0))


def _tc_pre(p0, p1, x_pad):
    def body(p0_ref, p1_ref, x_ref, dinv_ref, u0_ref):
        deg = p0_ref[:, :1] + p1_ref[:, :1] + 1.0
        dinv = lax.rsqrt(deg)
        dinv_ref[...] = dinv
        u0_ref[...] = x_ref[...] * dinv

    return pl.pallas_call(
        body,
        grid=(N_PAD // BR,),
        in_specs=_row_specs(16, 16, 128),
        out_specs=_row_specs(1, 128),
        out_shape=[jax.ShapeDtypeStruct((N_PAD, 1), jnp.float32),
                   jax.ShapeDtypeStruct((N_PAD, 128), jnp.float32)],
    )(p0, p1, x_pad)


def _tc_layer1(s0, s1, u0, dinv, W1, b1, W2a, W2b):
    def body(s0_ref, s1_ref, u_ref, d_ref, w1_ref, b1_ref, w2a_ref, w2b_ref,
             outa_ref, outb_ref):
        dinv = d_ref[...]
        g1 = dinv * (s0_ref[...] + s1_ref[...] + u_ref[...])
        h1 = jax.nn.relu(
            jnp.dot(g1, w1_ref[...], precision=_PREC,
                    preferred_element_type=jnp.float32) + b1_ref[...])
        outa_ref[...] = dinv * jnp.dot(h1, w2a_ref[...], precision=_PREC,
                                       preferred_element_type=jnp.float32)
        outb_ref[...] = dinv * jnp.dot(h1, w2b_ref[...], precision=_PREC,
                                       preferred_element_type=jnp.float32)

    fa, fb = W2a.shape[1], W2b.shape[1]
    return pl.pallas_call(
        body,
        grid=(N_PAD // BR,),
        in_specs=_row_specs(128, 128, 128, 1)
        + [_full_spec(W1.shape), _full_spec(b1.shape),
           _full_spec(W2a.shape), _full_spec(W2b.shape)],
        out_specs=_row_specs(fa, fb),
        out_shape=[jax.ShapeDtypeStruct((N_PAD, fa), jnp.float32),
                   jax.ShapeDtypeStruct((N_PAD, fb), jnp.float32)],
    )(s0, s1, u0, dinv, W1, b1, W2a, W2b)


def _tc_layer2(s0a, s1a, ua, s0b, s1b, ub, dinv, b2a, b2b, W3a, W3b):
    fa, fb = ua.shape[1], ub.shape[1]
    fo = W3a.shape[1]

    def body(s0a_ref, s1a_ref, ua_ref, s0b_ref, s1b_ref, ub_ref, d_ref,
             ba_ref, bb_ref, wa_ref, wb_ref, out_ref):
        dinv = d_ref[...]
        ga = dinv * (s0a_ref[...] + s1a_ref[...] + ua_ref[...])
        gb = dinv * (s0b_ref[...] + s1b_ref[...] + ub_ref[...])
        ha = jax.nn.relu(ga + ba_ref[...])
        hb = jax.nn.relu(gb + bb_ref[...])
        t = (jnp.dot(ha, wa_ref[...], precision=_PREC,
                     preferred_element_type=jnp.float32)
             + jnp.dot(hb, wb_ref[...], precision=_PREC,
                       preferred_element_type=jnp.float32))
        out_ref[...] = dinv * t

    return pl.pallas_call(
        body,
        grid=(N_PAD // BR,),
        in_specs=_row_specs(fa, fa, fa, fb, fb, fb, 1)
        + [_full_spec(b2a.shape), _full_spec(b2b.shape),
           _full_spec(W3a.shape), _full_spec(W3b.shape)],
        out_specs=_row_specs(fo)[0],
        out_shape=jax.ShapeDtypeStruct((N_PAD, fo), jnp.float32),
    )(s0a, s1a, ua, s0b, s1b, ub, dinv, b2a, b2b, W3a, W3b)


def _tc_layer(s0, s1, u, dinv, bprev, Wnext):
    fi = u.shape[1]
    fo = Wnext.shape[1]

    def body(s0_ref, s1_ref, u_ref, d_ref, b_ref, w_ref, out_ref):
        dinv = d_ref[...]
        g = dinv * (s0_ref[...] + s1_ref[...] + u_ref[...])
        h = jax.nn.relu(g + b_ref[...])
        t = jnp.dot(h, w_ref[...], precision=_PREC,
                    preferred_element_type=jnp.float32)
        out_ref[...] = dinv * t

    return pl.pallas_call(
        body,
        grid=(N_PAD // BR,),
        in_specs=_row_specs(fi, fi, fi, 1)
        + [_full_spec(bprev.shape), _full_spec(Wnext.shape)],
        out_specs=_row_specs(fo)[0],
        out_shape=jax.ShapeDtypeStruct((N_PAD, fo), jnp.float32),
    )(s0, s1, u, dinv, bprev, Wnext)


def _tc_layer4(s0, s1, u, dinv, b4p):
    fi = u.shape[1]

    def body(s0_ref, s1_ref, u_ref, d_ref, b_ref, out_ref):
        dinv = d_ref[...]
        g = dinv * (s0_ref[...] + s1_ref[...] + u_ref[...])
        out_ref[...] = dinv * jax.nn.relu(g + b_ref[...])

    return pl.pallas_call(
        body,
        grid=(N_PAD // BR,),
        in_specs=_row_specs(fi, fi, fi, 1) + [_full_spec(b4p.shape)],
        out_specs=_row_specs(fi)[0],
        out_shape=jax.ShapeDtypeStruct((N_PAD, fi), jnp.float32),
    )(s0, s1, u, dinv, b4p)


def _tc_head(s0, s1, u5, dinv, Wmu, bmu, Wls, bls, eps_pad):
    fi = u5.shape[1]

    def body(s0_ref, s1_ref, u_ref, d_ref, wmu_ref, bmu_ref, wls_ref,
             bls_ref, eps_ref, pz_ref, z_ref):
        g = d_ref[...] * (s0_ref[...] + s1_ref[...] + u_ref[...])
        mu = jnp.dot(g, wmu_ref[...], precision=_PREC,
                     preferred_element_type=jnp.float32) + bmu_ref[...]
        ls = jnp.dot(g, wls_ref[...], precision=_PREC,
                     preferred_element_type=jnp.float32) + bls_ref[...]
        z = mu + eps_ref[...] * jnp.exp(ls)
        m = jnp.max(z, axis=1, keepdims=True)
        pz = z - m - jnp.log(jnp.sum(jnp.exp(z - m), axis=1, keepdims=True))
        pz_ref[...] = pz
        z_ref[...] = z

    return pl.pallas_call(
        body,
        grid=(N_PAD // BR,),
        in_specs=_row_specs(fi, fi, fi, 1)
        + [_full_spec(Wmu.shape), _full_spec(bmu.shape),
           _full_spec(Wls.shape), _full_spec(bls.shape)]
        + _row_specs(21),
        out_specs=_row_specs(21, 21),
        out_shape=[jax.ShapeDtypeStruct((N_PAD, 21), jnp.float32),
                   jax.ShapeDtypeStruct((N_PAD, 21), jnp.float32)],
    )(s0, s1, u5, dinv, Wmu, bmu, Wls, bls, eps_pad)


def _pad2(a, rows, cols):
    return jnp.pad(a, ((0, rows - a.shape[0]), (0, cols - a.shape[1])))


def kernel(x, edge_index, eps, W1, b1, W2, b2, W3, b3, W4, b4,
           W_mu, b_mu, W_ls, b_ls):
    src = edge_index[0]
    dst = edge_index[1]
    pad = E_PAD - N_EDGES
    fill = jnp.full((pad,), N_NODES, dtype=jnp.int32)
    src_p = jnp.concatenate([src, fill]).reshape(NW, CH, K)
    dst_p = jnp.concatenate([dst, fill]).reshape(NW, CH, K)

    x_pad = jnp.pad(x, ((0, N_PAD - N_NODES), (0, 0)))
    eps_pad = jnp.pad(eps, ((0, N_PAD - N_NODES), (0, 0)))

    W2a = W2[:, :128]
    W2b = _pad2(W2[:, 128:], 336, 48)
    b2a = b2[:128].reshape(1, 128)
    b2b = jnp.pad(b2[128:], (0, 8)).reshape(1, 48)
    W3a = jnp.pad(W3[:128, :], ((0, 0), (0, 12)))
    W3b = _pad2(W3[128:, :], 48, 96)
    b3p = jnp.pad(b3, (0, 12)).reshape(1, 96)
    W4p = _pad2(W4, 96, 48)
    b4p = jnp.pad(b4, (0, 6)).reshape(1, 48)
    Wmup = _pad2(W_mu, 48, 21)
    Wlsp = _pad2(W_ls, 48, 21)
    b1r = b1.reshape(1, 336)
    bmur = b_mu.reshape(1, 21)
    blsr = b_ls.reshape(1, 21)

    p = _sc_degree(dst_p)
    dinv, u0 = _tc_pre(p[0], p[1], x_pad)

    s = _make_prop(128)(u0, src_p, dst_p)
    u2a, u2b = _tc_layer1(s[0], s[1], u0, dinv, W1, b1r, W2a, W2b)

    sa = _make_prop(128)(u2a, src_p, dst_p)
    sb = _make_prop(48)(u2b, src_p, dst_p)
    u3 = _tc_layer2(sa[0], sa[1], u2a, sb[0], sb[1], u2b, dinv,
                    b2a, b2b, W3a, W3b)

    s = _make_prop(96)(u3, src_p, dst_p)
    u4 = _tc_layer(s[0], s[1], u3, dinv, b3p, W4p)

    s = _make_prop(48)(u4, src_p, dst_p)
    u5 = _tc_layer4(s[0], s[1], u4, dinv, b4p)

    s = _make_prop(48)(u5, src_p, dst_p)
    pz, z = _tc_head(s[0], s[1], u5, dinv, Wmup, bmur, Wlsp, blsr, eps_pad)

    return (pz[:N_NODES], z[:N_NODES])


# speedup vs baseline: 21.5033x; 1.8139x over previous
import functools

import jax
import jax.numpy as jnp
from jax import lax
from jax.experimental import pallas as pl
from jax.experimental.pallas import tpu as pltpu
from jax.experimental.pallas import tpu_sc as plsc

N_NODES = 10000
N_EDGES = 320000
N_PAD = 10240
NW = 32
NSC = 16
K = 96
CH = 106
E_PAD = NW * CH * K
RPS = N_PAD // NSC
BR = 1280

_MESH = plsc.VectorSubcoreMesh(core_axis_name="c", subcore_axis_name="s")
_PREC = jax.lax.Precision.HIGHEST
_SC_PARAMS = pltpu.CompilerParams(use_tc_tiling_on_sc=False)


def _zero_fill(stage, f):
    zeros = jnp.zeros((16,), jnp.float32)

    @pl.loop(0, K)
    def _(r):
        for cc in range(f // 16):
            stage[r, pl.ds(cc * 16, 16)] = zeros


def _stripe_zero(stage, acc, s):
    for b in range(RPS // K):
        pltpu.sync_copy(stage, acc.at[pl.ds(s * RPS + b * K, K)])
    rem = RPS % K
    if rem:
        pltpu.sync_copy(stage.at[pl.ds(0, rem)],
                        acc.at[pl.ds(s * RPS + (RPS // K) * K, rem)])


@functools.partial(
    pl.kernel,
    out_type=jax.ShapeDtypeStruct((2, N_PAD, 16), jnp.float32),
    mesh=_MESH,
    scratch_types=[
        pltpu.VMEM((CH, K), jnp.int32),
        pltpu.VMEM((K, 16), jnp.float32),
        pltpu.VMEM_SHARED((N_PAD, 16), jnp.float32),
        pltpu.SemaphoreType.DMA,
    ],
    compiler_params=_SC_PARAMS,
)
def _sc_degree(dst_hbm, out_hbm, dst_v, stage, acc, sem):
    c = lax.axis_index("c")
    s = lax.axis_index("s")
    wid = c * NSC + s
    pltpu.sync_copy(dst_hbm.at[wid], dst_v)
    _zero_fill(stage, 16)
    _stripe_zero(stage, acc, s)
    basis = jnp.where(lax.iota(jnp.int32, 16) == 0, 1.0, 0.0)

    @pl.loop(0, K)
    def _(r):
        stage[r, pl.ds(0, 16)] = basis

    plsc.subcore_barrier()

    @pl.loop(0, CH)
    def _(j):
        pltpu.async_copy(stage, acc.at[dst_v.at[j]], sem, add=True)

        @pl.when(j >= 7)
        def _():
            pltpu.make_async_copy(stage, acc.at[dst_v.at[j]], sem).wait()

    @pl.loop(0, 7)
    def _(j):
        pltpu.make_async_copy(stage, acc.at[dst_v.at[j]], sem).wait()

    plsc.subcore_barrier()
    pltpu.sync_copy(acc.at[pl.ds(s * RPS, RPS)],
                    out_hbm.at[c, pl.ds(s * RPS, RPS)])


@functools.lru_cache(maxsize=None)
def _make_prop(f):
    @functools.partial(
        pl.kernel,
        out_type=jax.ShapeDtypeStruct((2, N_PAD, f), jnp.float32),
        mesh=_MESH,
        scratch_types=[
            pltpu.VMEM((CH, K), jnp.int32),
            pltpu.VMEM((CH, K), jnp.int32),
            pltpu.VMEM((K, f), jnp.float32),
            pltpu.VMEM((K, f), jnp.float32),
            pltpu.VMEM_SHARED((N_PAD, f), jnp.float32),
            pltpu.SemaphoreType.DMA,
            pltpu.SemaphoreType.DMA,
        ],
        compiler_params=_SC_PARAMS,
    )
    def prop(u_hbm, src_hbm, dst_hbm, out_hbm, src_v, dst_v, st0, st1, acc,
             sem0, sem1):
        c = lax.axis_index("c")
        s = lax.axis_index("s")
        wid = c * NSC + s
        pltpu.sync_copy(src_hbm.at[wid], src_v)
        pltpu.sync_copy(dst_hbm.at[wid], dst_v)
        _zero_fill(st0, f)
        _stripe_zero(st0, acc, s)
        plsc.subcore_barrier()

        pltpu.async_copy(u_hbm.at[src_v.at[0]], st0, sem0)

        @pl.loop(0, CH // 2)
        def _(k):
            j0 = 2 * k
            j1 = j0 + 1
            pltpu.make_async_copy(u_hbm.at[src_v.at[j0]], st0, sem0).wait()
            pltpu.async_copy(u_hbm.at[src_v.at[j1]], st1, sem1)
            pltpu.sync_copy(st0, acc.at[dst_v.at[j0]], add=True)
            pltpu.make_async_copy(u_hbm.at[src_v.at[j1]], st1, sem1).wait()

            @pl.when(j1 + 1 < CH)
            def _():
                pltpu.async_copy(u_hbm.at[src_v.at[j1 + 1]], st0, sem0)

            pltpu.sync_copy(st1, acc.at[dst_v.at[j1]], add=True)

        if CH % 2:
            pltpu.make_async_copy(u_hbm.at[src_v.at[CH - 1]], st0, sem0).wait()
            pltpu.sync_copy(st0, acc.at[dst_v.at[CH - 1]], add=True)

        plsc.subcore_barrier()
        pltpu.sync_copy(acc.at[pl.ds(s * RPS, RPS)],
                        out_hbm.at[c, pl.ds(s * RPS, RPS)])

    return prop


def _row_specs(*widths):
    return [pl.BlockSpec((BR, w), lambda i: (i, 0)) for w in widths]


def _full_spec(shape):
    return pl.BlockSpec(shape, lambda i: (0, 0))


def _tc_pre(p0, p1, x_pad):
    def body(p0_ref, p1_ref, x_ref, dinv_ref, u0_ref):
        deg = p0_ref[:, :1] + p1_ref[:, :1] + 1.0
        dinv = lax.rsqrt(deg)
        dinv_ref[...] = dinv
        u0_ref[...] = x_ref[...] * dinv

    return pl.pallas_call(
        body,
        grid=(N_PAD // BR,),
        in_specs=_row_specs(16, 16, 128),
        out_specs=_row_specs(1, 128),
        out_shape=[jax.ShapeDtypeStruct((N_PAD, 1), jnp.float32),
                   jax.ShapeDtypeStruct((N_PAD, 128), jnp.float32)],
    )(p0, p1, x_pad)


def _tc_layer1(s0, s1, u0, dinv, W1, b1, W2a, W2b):
    def body(s0_ref, s1_ref, u_ref, d_ref, w1_ref, b1_ref, w2a_ref, w2b_ref,
             outa_ref, outb_ref):
        dinv = d_ref[...]
        g1 = dinv * (s0_ref[...] + s1_ref[...] + u_ref[...])
        h1 = jax.nn.relu(
            jnp.dot(g1, w1_ref[...], precision=_PREC,
                    preferred_element_type=jnp.float32) + b1_ref[...])
        outa_ref[...] = dinv * jnp.dot(h1, w2a_ref[...], precision=_PREC,
                                       preferred_element_type=jnp.float32)
        outb_ref[...] = dinv * jnp.dot(h1, w2b_ref[...], precision=_PREC,
                                       preferred_element_type=jnp.float32)

    fa, fb = W2a.shape[1], W2b.shape[1]
    return pl.pallas_call(
        body,
        grid=(N_PAD // BR,),
        in_specs=_row_specs(128, 128, 128, 1)
        + [_full_spec(W1.shape), _full_spec(b1.shape),
           _full_spec(W2a.shape), _full_spec(W2b.shape)],
        out_specs=_row_specs(fa, fb),
        out_shape=[jax.ShapeDtypeStruct((N_PAD, fa), jnp.float32),
                   jax.ShapeDtypeStruct((N_PAD, fb), jnp.float32)],
    )(s0, s1, u0, dinv, W1, b1, W2a, W2b)


def _tc_layer2(s0a, s1a, ua, s0b, s1b, ub, dinv, b2a, b2b, W3a, W3b):
    fa, fb = ua.shape[1], ub.shape[1]
    fo = W3a.shape[1]

    def body(s0a_ref, s1a_ref, ua_ref, s0b_ref, s1b_ref, ub_ref, d_ref,
             ba_ref, bb_ref, wa_ref, wb_ref, out_ref):
        dinv = d_ref[...]
        ga = dinv * (s0a_ref[...] + s1a_ref[...] + ua_ref[...])
        gb = dinv * (s0b_ref[...] + s1b_ref[...] + ub_ref[...])
        ha = jax.nn.relu(ga + ba_ref[...])
        hb = jax.nn.relu(gb + bb_ref[...])
        t = (jnp.dot(ha, wa_ref[...], precision=_PREC,
                     preferred_element_type=jnp.float32)
             + jnp.dot(hb, wb_ref[...], precision=_PREC,
                       preferred_element_type=jnp.float32))
        out_ref[...] = dinv * t

    return pl.pallas_call(
        body,
        grid=(N_PAD // BR,),
        in_specs=_row_specs(fa, fa, fa, fb, fb, fb, 1)
        + [_full_spec(b2a.shape), _full_spec(b2b.shape),
           _full_spec(W3a.shape), _full_spec(W3b.shape)],
        out_specs=_row_specs(fo)[0],
        out_shape=jax.ShapeDtypeStruct((N_PAD, fo), jnp.float32),
    )(s0a, s1a, ua, s0b, s1b, ub, dinv, b2a, b2b, W3a, W3b)


def _tc_layer(s0, s1, u, dinv, bprev, Wnext):
    fi = u.shape[1]
    fo = Wnext.shape[1]

    def body(s0_ref, s1_ref, u_ref, d_ref, b_ref, w_ref, out_ref):
        dinv = d_ref[...]
        g = dinv * (s0_ref[...] + s1_ref[...] + u_ref[...])
        h = jax.nn.relu(g + b_ref[...])
        t = jnp.dot(h, w_ref[...], precision=_PREC,
                    preferred_element_type=jnp.float32)
        out_ref[...] = dinv * t

    return pl.pallas_call(
        body,
        grid=(N_PAD // BR,),
        in_specs=_row_specs(fi, fi, fi, 1)
        + [_full_spec(bprev.shape), _full_spec(Wnext.shape)],
        out_specs=_row_specs(fo)[0],
        out_shape=jax.ShapeDtypeStruct((N_PAD, fo), jnp.float32),
    )(s0, s1, u, dinv, bprev, Wnext)


def _tc_layer4(s0, s1, u, dinv, b4p):
    fi = u.shape[1]

    def body(s0_ref, s1_ref, u_ref, d_ref, b_ref, out_ref):
        dinv = d_ref[...]
        g = dinv * (s0_ref[...] + s1_ref[...] + u_ref[...])
        out_ref[...] = dinv * jax.nn.relu(g + b_ref[...])

    return pl.pallas_call(
        body,
        grid=(N_PAD // BR,),
        in_specs=_row_specs(fi, fi, fi, 1) + [_full_spec(b4p.shape)],
        out_specs=_row_specs(fi)[0],
        out_shape=jax.ShapeDtypeStruct((N_PAD, fi), jnp.float32),
    )(s0, s1, u, dinv, b4p)


def _tc_head(s0, s1, u5, dinv, Wmu, bmu, Wls, bls, eps_pad):
    fi = u5.shape[1]

    def body(s0_ref, s1_ref, u_ref, d_ref, wmu_ref, bmu_ref, wls_ref,
             bls_ref, eps_ref, pz_ref, z_ref):
        g = d_ref[...] * (s0_ref[...] + s1_ref[...] + u_ref[...])
        mu = jnp.dot(g, wmu_ref[...], precision=_PREC,
                     preferred_element_type=jnp.float32) + bmu_ref[...]
        ls = jnp.dot(g, wls_ref[...], precision=_PREC,
                     preferred_element_type=jnp.float32) + bls_ref[...]
        z = mu + eps_ref[...] * jnp.exp(ls)
        m = jnp.max(z, axis=1, keepdims=True)
        pz = z - m - jnp.log(jnp.sum(jnp.exp(z - m), axis=1, keepdims=True))
        pz_ref[...] = pz
        z_ref[...] = z

    return pl.pallas_call(
        body,
        grid=(N_PAD // BR,),
        in_specs=_row_specs(fi, fi, fi, 1)
        + [_full_spec(Wmu.shape), _full_spec(bmu.shape),
           _full_spec(Wls.shape), _full_spec(bls.shape)]
        + _row_specs(21),
        out_specs=_row_specs(21, 21),
        out_shape=[jax.ShapeDtypeStruct((N_PAD, 21), jnp.float32),
                   jax.ShapeDtypeStruct((N_PAD, 21), jnp.float32)],
    )(s0, s1, u5, dinv, Wmu, bmu, Wls, bls, eps_pad)


def _pad2(a, rows, cols):
    return jnp.pad(a, ((0, rows - a.shape[0]), (0, cols - a.shape[1])))


def kernel(x, edge_index, eps, W1, b1, W2, b2, W3, b3, W4, b4,
           W_mu, b_mu, W_ls, b_ls):
    src = edge_index[0]
    dst = edge_index[1]
    pad = E_PAD - N_EDGES
    fill = N_NODES + jnp.arange(pad, dtype=jnp.int32) % (N_PAD - N_NODES)
    src_p = jnp.concatenate([src, fill]).reshape(NW, CH, K)
    dst_p = jnp.concatenate([dst, fill]).reshape(NW, CH, K)

    x_pad = jnp.pad(x, ((0, N_PAD - N_NODES), (0, 0)))
    eps_pad = jnp.pad(eps, ((0, N_PAD - N_NODES), (0, 0)))

    W2a = W2[:, :128]
    W2b = _pad2(W2[:, 128:], 336, 48)
    b2a = b2[:128].reshape(1, 128)
    b2b = jnp.pad(b2[128:], (0, 8)).reshape(1, 48)
    W3a = jnp.pad(W3[:128, :], ((0, 0), (0, 12)))
    W3b = _pad2(W3[128:, :], 48, 96)
    b3p = jnp.pad(b3, (0, 12)).reshape(1, 96)
    W4p = _pad2(W4, 96, 48)
    b4p = jnp.pad(b4, (0, 6)).reshape(1, 48)
    Wmup = _pad2(W_mu, 48, 21)
    Wlsp = _pad2(W_ls, 48, 21)
    b1r = b1.reshape(1, 336)
    bmur = b_mu.reshape(1, 21)
    blsr = b_ls.reshape(1, 21)

    p = _sc_degree(dst_p)
    dinv, u0 = _tc_pre(p[0], p[1], x_pad)

    s = _make_prop(128)(u0, src_p, dst_p)
    u2a, u2b = _tc_layer1(s[0], s[1], u0, dinv, W1, b1r, W2a, W2b)

    sa = _make_prop(128)(u2a, src_p, dst_p)
    sb = _make_prop(48)(u2b, src_p, dst_p)
    u3 = _tc_layer2(sa[0], sa[1], u2a, sb[0], sb[1], u2b, dinv,
                    b2a, b2b, W3a, W3b)

    s = _make_prop(96)(u3, src_p, dst_p)
    u4 = _tc_layer(s[0], s[1], u3, dinv, b3p, W4p)

    s = _make_prop(48)(u4, src_p, dst_p)
    u5 = _tc_layer4(s[0], s[1], u4, dinv, b4p)

    s = _make_prop(48)(u5, src_p, dst_p)
    pz, z = _tc_head(s[0], s[1], u5, dinv, Wmup, bmur, Wlsp, blsr, eps_pad)

    return (pz[:N_NODES], z[:N_NODES])


# speedup vs baseline: 26.1429x; 1.2158x over previous
import functools

import jax
import jax.numpy as jnp
from jax import lax
from jax.experimental import pallas as pl
from jax.experimental.pallas import tpu as pltpu
from jax.experimental.pallas import tpu_sc as plsc

N_NODES = 10000
N_EDGES = 320000
N_PAD = 10240
NW = 32
NSC = 16
K = 96
CH = 108
E_PAD = NW * CH * K
RPS = N_PAD // NSC
BR = 1280

_MESH = plsc.VectorSubcoreMesh(core_axis_name="c", subcore_axis_name="s")
_PREC = jax.lax.Precision.HIGHEST
_SC_PARAMS = pltpu.CompilerParams(use_tc_tiling_on_sc=False)


def _zero_fill(stage, f):
    zeros = jnp.zeros((16,), jnp.float32)

    @pl.loop(0, K)
    def _(r):
        for cc in range(f // 16):
            stage[r, pl.ds(cc * 16, 16)] = zeros


def _stripe_zero(stage, acc, s):
    for b in range(RPS // K):
        pltpu.sync_copy(stage, acc.at[pl.ds(s * RPS + b * K, K)])
    rem = RPS % K
    if rem:
        pltpu.sync_copy(stage.at[pl.ds(0, rem)],
                        acc.at[pl.ds(s * RPS + (RPS // K) * K, rem)])


@functools.partial(
    pl.kernel,
    out_type=jax.ShapeDtypeStruct((2, N_PAD, 16), jnp.float32),
    mesh=_MESH,
    scratch_types=[
        pltpu.VMEM((CH, K), jnp.int32),
        pltpu.VMEM((K, 16), jnp.float32),
        pltpu.VMEM_SHARED((N_PAD, 16), jnp.float32),
        pltpu.SemaphoreType.DMA,
    ],
    compiler_params=_SC_PARAMS,
)
def _sc_degree(dst_hbm, out_hbm, dst_v, stage, acc, sem):
    c = lax.axis_index("c")
    s = lax.axis_index("s")
    wid = c * NSC + s
    pltpu.sync_copy(dst_hbm.at[wid], dst_v)
    _zero_fill(stage, 16)
    _stripe_zero(stage, acc, s)
    basis = jnp.where(lax.iota(jnp.int32, 16) == 0, 1.0, 0.0)

    @pl.loop(0, K)
    def _(r):
        stage[r, pl.ds(0, 16)] = basis

    plsc.subcore_barrier()

    @pl.loop(0, CH)
    def _(j):
        pltpu.async_copy(stage, acc.at[dst_v.at[j]], sem, add=True)

        @pl.when(j >= 7)
        def _():
            pltpu.make_async_copy(stage, acc.at[dst_v.at[j]], sem).wait()

    @pl.loop(0, 7)
    def _(j):
        pltpu.make_async_copy(stage, acc.at[dst_v.at[j]], sem).wait()

    plsc.subcore_barrier()
    pltpu.sync_copy(acc.at[pl.ds(s * RPS, RPS)],
                    out_hbm.at[c, pl.ds(s * RPS, RPS)])


@functools.lru_cache(maxsize=None)
def _make_prop(f, nb):
    assert CH % nb == 0

    @functools.partial(
        pl.kernel,
        out_type=jax.ShapeDtypeStruct((2, N_PAD, f), jnp.float32),
        mesh=_MESH,
        scratch_types=[
            pltpu.VMEM((CH, K), jnp.int32),
            pltpu.VMEM((CH, K), jnp.int32),
            [pltpu.VMEM((K, f), jnp.float32) for _ in range(nb)],
            pltpu.VMEM_SHARED((N_PAD, f), jnp.float32),
            [pltpu.SemaphoreType.DMA for _ in range(nb)],
            [pltpu.SemaphoreType.DMA for _ in range(nb)],
        ],
        compiler_params=_SC_PARAMS,
    )
    def prop(u_hbm, src_hbm, dst_hbm, out_hbm, src_v, dst_v, st, acc,
             sem_g, sem_s):
        c = lax.axis_index("c")
        s = lax.axis_index("s")
        wid = c * NSC + s
        pltpu.sync_copy(src_hbm.at[wid], src_v)
        pltpu.sync_copy(dst_hbm.at[wid], dst_v)
        _zero_fill(st[0], f)
        _stripe_zero(st[0], acc, s)
        plsc.subcore_barrier()

        for b in range(nb):
            pltpu.async_copy(u_hbm.at[src_v.at[b]], st[b], sem_g[b])

        @pl.loop(0, CH // nb - 1)
        def _(g):
            base = g * nb
            for b in range(nb):
                j = base + b
                pltpu.make_async_copy(u_hbm.at[src_v.at[j]], st[b],
                                      sem_g[b]).wait()
                pltpu.async_copy(st[b], acc.at[dst_v.at[j]], sem_s[b],
                                 add=True)
            for b in range(nb):
                j = base + b
                pltpu.make_async_copy(st[b], acc.at[dst_v.at[j]],
                                      sem_s[b]).wait()
                pltpu.async_copy(u_hbm.at[src_v.at[j + nb]], st[b], sem_g[b])

        base = CH - nb
        for b in range(nb):
            j = base + b
            pltpu.make_async_copy(u_hbm.at[src_v.at[j]], st[b],
                                  sem_g[b]).wait()
            pltpu.async_copy(st[b], acc.at[dst_v.at[j]], sem_s[b], add=True)
        for b in range(nb):
            j = base + b
            pltpu.make_async_copy(st[b], acc.at[dst_v.at[j]], sem_s[b]).wait()

        plsc.subcore_barrier()
        pltpu.sync_copy(acc.at[pl.ds(s * RPS, RPS)],
                        out_hbm.at[c, pl.ds(s * RPS, RPS)])

    return prop


def _row_specs(*widths):
    return [pl.BlockSpec((BR, w), lambda i: (i, 0)) for w in widths]


def _full_spec(shape):
    return pl.BlockSpec(shape, lambda i: (0, 0))


def _tc_pre(p0, p1, x_pad):
    def body(p0_ref, p1_ref, x_ref, dinv_ref, u0_ref):
        deg = p0_ref[...] + p1_ref[...] + 1.0
        dinv = lax.rsqrt(deg)
        dinv_ref[...] = dinv
        u0_ref[...] = x_ref[...] * dinv

    return pl.pallas_call(
        body,
        grid=(N_PAD // BR,),
        in_specs=_row_specs(1, 1, 128),
        out_specs=_row_specs(1, 128),
        out_shape=[jax.ShapeDtypeStruct((N_PAD, 1), jnp.float32),
                   jax.ShapeDtypeStruct((N_PAD, 128), jnp.float32)],
    )(p0, p1, x_pad)


def _tc_layer1(s0, s1, u0, dinv, W1, b1, W2a, W2b):
    def body(s0_ref, s1_ref, u_ref, d_ref, w1_ref, b1_ref, w2a_ref, w2b_ref,
             outa_ref, outb_ref):
        dinv = d_ref[...]
        g1 = dinv * (s0_ref[...] + s1_ref[...] + u_ref[...])
        h1 = jax.nn.relu(
            jnp.dot(g1, w1_ref[...], precision=_PREC,
                    preferred_element_type=jnp.float32) + b1_ref[...])
        outa_ref[...] = dinv * jnp.dot(h1, w2a_ref[...], precision=_PREC,
                                       preferred_element_type=jnp.float32)
        outb_ref[...] = dinv * jnp.dot(h1, w2b_ref[...], precision=_PREC,
                                       preferred_element_type=jnp.float32)

    fa, fb = W2a.shape[1], W2b.shape[1]
    return pl.pallas_call(
        body,
        grid=(N_PAD // BR,),
        in_specs=_row_specs(128, 128, 128, 1)
        + [_full_spec(W1.shape), _full_spec(b1.shape),
           _full_spec(W2a.shape), _full_spec(W2b.shape)],
        out_specs=_row_specs(fa, fb),
        out_shape=[jax.ShapeDtypeStruct((N_PAD, fa), jnp.float32),
                   jax.ShapeDtypeStruct((N_PAD, fb), jnp.float32)],
    )(s0, s1, u0, dinv, W1, b1, W2a, W2b)


def _tc_layer2(s0a, s1a, ua, s0b, s1b, ub, dinv, b2a, b2b, W3a, W3b):
    fa, fb = ua.shape[1], ub.shape[1]
    fo = W3a.shape[1]

    def body(s0a_ref, s1a_ref, ua_ref, s0b_ref, s1b_ref, ub_ref, d_ref,
             ba_ref, bb_ref, wa_ref, wb_ref, out_ref):
        dinv = d_ref[...]
        ga = dinv * (s0a_ref[...] + s1a_ref[...] + ua_ref[...])
        gb = dinv * (s0b_ref[...] + s1b_ref[...] + ub_ref[...])
        ha = jax.nn.relu(ga + ba_ref[...])
        hb = jax.nn.relu(gb + bb_ref[...])
        t = (jnp.dot(ha, wa_ref[...], precision=_PREC,
                     preferred_element_type=jnp.float32)
             + jnp.dot(hb, wb_ref[...], precision=_PREC,
                       preferred_element_type=jnp.float32))
        out_ref[...] = dinv * t

    return pl.pallas_call(
        body,
        grid=(N_PAD // BR,),
        in_specs=_row_specs(fa, fa, fa, fb, fb, fb, 1)
        + [_full_spec(b2a.shape), _full_spec(b2b.shape),
           _full_spec(W3a.shape), _full_spec(W3b.shape)],
        out_specs=_row_specs(fo)[0],
        out_shape=jax.ShapeDtypeStruct((N_PAD, fo), jnp.float32),
    )(s0a, s1a, ua, s0b, s1b, ub, dinv, b2a, b2b, W3a, W3b)


def _tc_layer(s0, s1, u, dinv, bprev, Wnext):
    fi = u.shape[1]
    fo = Wnext.shape[1]

    def body(s0_ref, s1_ref, u_ref, d_ref, b_ref, w_ref, out_ref):
        dinv = d_ref[...]
        g = dinv * (s0_ref[...] + s1_ref[...] + u_ref[...])
        h = jax.nn.relu(g + b_ref[...])
        t = jnp.dot(h, w_ref[...], precision=_PREC,
                    preferred_element_type=jnp.float32)
        out_ref[...] = dinv * t

    return pl.pallas_call(
        body,
        grid=(N_PAD // BR,),
        in_specs=_row_specs(fi, fi, fi, 1)
        + [_full_spec(bprev.shape), _full_spec(Wnext.shape)],
        out_specs=_row_specs(fo)[0],
        out_shape=jax.ShapeDtypeStruct((N_PAD, fo), jnp.float32),
    )(s0, s1, u, dinv, bprev, Wnext)


def _tc_layer4(s0, s1, u, dinv, b4p):
    fi = u.shape[1]

    def body(s0_ref, s1_ref, u_ref, d_ref, b_ref, out_ref):
        dinv = d_ref[...]
        g = dinv * (s0_ref[...] + s1_ref[...] + u_ref[...])
        out_ref[...] = dinv * jax.nn.relu(g + b_ref[...])

    return pl.pallas_call(
        body,
        grid=(N_PAD // BR,),
        in_specs=_row_specs(fi, fi, fi, 1) + [_full_spec(b4p.shape)],
        out_specs=_row_specs(fi)[0],
        out_shape=jax.ShapeDtypeStruct((N_PAD, fi), jnp.float32),
    )(s0, s1, u, dinv, b4p)


def _tc_head(s0, s1, u5, dinv, Wmu, bmu, Wls, bls, eps_pad):
    fi = u5.shape[1]

    def body(s0_ref, s1_ref, u_ref, d_ref, wmu_ref, bmu_ref, wls_ref,
             bls_ref, eps_ref, pz_ref, z_ref):
        g = d_ref[...] * (s0_ref[...] + s1_ref[...] + u_ref[...])
        mu = jnp.dot(g, wmu_ref[...], precision=_PREC,
                     preferred_element_type=jnp.float32) + bmu_ref[...]
        ls = jnp.dot(g, wls_ref[...], precision=_PREC,
                     preferred_element_type=jnp.float32) + bls_ref[...]
        z = mu + eps_ref[...] * jnp.exp(ls)
        m = jnp.max(z, axis=1, keepdims=True)
        pz = z - m - jnp.log(jnp.sum(jnp.exp(z - m), axis=1, keepdims=True))
        pz_ref[...] = pz
        z_ref[...] = z

    return pl.pallas_call(
        body,
        grid=(N_PAD // BR,),
        in_specs=_row_specs(fi, fi, fi, 1)
        + [_full_spec(Wmu.shape), _full_spec(bmu.shape),
           _full_spec(Wls.shape), _full_spec(bls.shape)]
        + _row_specs(21),
        out_specs=_row_specs(21, 21),
        out_shape=[jax.ShapeDtypeStruct((N_PAD, 21), jnp.float32),
                   jax.ShapeDtypeStruct((N_PAD, 21), jnp.float32)],
    )(s0, s1, u5, dinv, Wmu, bmu, Wls, bls, eps_pad)


def _pad2(a, rows, cols):
    return jnp.pad(a, ((0, rows - a.shape[0]), (0, cols - a.shape[1])))


def kernel(x, edge_index, eps, W1, b1, W2, b2, W3, b3, W4, b4,
           W_mu, b_mu, W_ls, b_ls):
    src = edge_index[0]
    dst = edge_index[1]
    pad = E_PAD - N_EDGES
    fill = N_NODES + jnp.arange(pad, dtype=jnp.int32) % (N_PAD - N_NODES)
    src_p = jnp.concatenate([src, fill]).reshape(NW, CH, K)
    dst_p = jnp.concatenate([dst, fill]).reshape(NW, CH, K)

    x_pad = jnp.pad(x, ((0, N_PAD - N_NODES), (0, 0)))
    eps_pad = jnp.pad(eps, ((0, N_PAD - N_NODES), (0, 0)))

    W2a = W2[:, :128]
    W2b = _pad2(W2[:, 128:], 336, 48)
    b2a = b2[:128].reshape(1, 128)
    b2b = jnp.pad(b2[128:], (0, 8)).reshape(1, 48)
    W3a = jnp.pad(W3[:128, :], ((0, 0), (0, 12)))
    W3b = _pad2(W3[128:, :], 48, 96)
    b3p = jnp.pad(b3, (0, 12)).reshape(1, 96)
    W4p = _pad2(W4, 96, 48)
    b4p = jnp.pad(b4, (0, 6)).reshape(1, 48)
    Wmup = _pad2(W_mu, 48, 21)
    Wlsp = _pad2(W_ls, 48, 21)
    b1r = b1.reshape(1, 336)
    bmur = b_mu.reshape(1, 21)
    blsr = b_ls.reshape(1, 21)

    p = _sc_degree(dst_p)
    dinv, u0 = _tc_pre(p[0, :, :1], p[1, :, :1], x_pad)

    s = _make_prop(128, 2)(u0, src_p, dst_p)
    u2a, u2b = _tc_layer1(s[0], s[1], u0, dinv, W1, b1r, W2a, W2b)

    sa = _make_prop(128, 2)(u2a, src_p, dst_p)
    sb = _make_prop(48, 6)(u2b, src_p, dst_p)
    u3 = _tc_layer2(sa[0], sa[1], u2a, sb[0], sb[1], u2b, dinv,
                    b2a, b2b, W3a, W3b)

    s = _make_prop(96, 4)(u3, src_p, dst_p)
    u4 = _tc_layer(s[0], s[1], u3, dinv, b3p, W4p)

    s = _make_prop(48, 6)(u4, src_p, dst_p)
    u5 = _tc_layer4(s[0], s[1], u4, dinv, b4p)

    s = _make_prop(48, 6)(u5, src_p, dst_p)
    pz, z = _tc_head(s[0], s[1], u5, dinv, Wmup, bmur, Wlsp, blsr, eps_pad)

    return (pz[:N_NODES], z[:N_NODES])


# speedup vs baseline: 28.3342x; 1.0838x over previous
import functools

import jax
import jax.numpy as jnp
from jax import lax
from jax.experimental import pallas as pl
from jax.experimental.pallas import tpu as pltpu
from jax.experimental.pallas import tpu_sc as plsc

N_NODES = 10000
N_EDGES = 320000
N_PAD = 10240
NW = 32
NSC = 16
K = 48
CH = 216
E_PAD = NW * CH * K
RPS = N_PAD // NSC
BR = 1280

_MESH = plsc.VectorSubcoreMesh(core_axis_name="c", subcore_axis_name="s")
_PREC = jax.lax.Precision.HIGHEST
_SC_PARAMS = pltpu.CompilerParams(use_tc_tiling_on_sc=False)


def _zero_fill(stage, f):
    zeros = jnp.zeros((16,), jnp.float32)

    @pl.loop(0, K)
    def _(r):
        for cc in range(f // 16):
            stage[r, pl.ds(cc * 16, 16)] = zeros


def _stripe_zero(stage, acc, s):
    for b in range(RPS // K):
        pltpu.sync_copy(stage, acc.at[pl.ds(s * RPS + b * K, K)])
    rem = RPS % K
    if rem:
        pltpu.sync_copy(stage.at[pl.ds(0, rem)],
                        acc.at[pl.ds(s * RPS + (RPS // K) * K, rem)])


@functools.partial(
    pl.kernel,
    out_type=jax.ShapeDtypeStruct((2, N_PAD, 16), jnp.float32),
    mesh=_MESH,
    scratch_types=[
        pltpu.VMEM((CH, K), jnp.int32),
        pltpu.VMEM((K, 16), jnp.float32),
        pltpu.VMEM_SHARED((N_PAD, 16), jnp.float32),
        pltpu.SemaphoreType.DMA,
    ],
    compiler_params=_SC_PARAMS,
)
def _sc_degree(dst_hbm, out_hbm, dst_v, stage, acc, sem):
    c = lax.axis_index("c")
    s = lax.axis_index("s")
    wid = c * NSC + s
    pltpu.sync_copy(dst_hbm.at[wid], dst_v)
    _zero_fill(stage, 16)
    _stripe_zero(stage, acc, s)
    basis = jnp.where(lax.iota(jnp.int32, 16) == 0, 1.0, 0.0)

    @pl.loop(0, K)
    def _(r):
        stage[r, pl.ds(0, 16)] = basis

    plsc.subcore_barrier()

    @pl.loop(0, CH)
    def _(j):
        pltpu.async_copy(stage, acc.at[dst_v.at[j]], sem, add=True)

        @pl.when(j >= 7)
        def _():
            pltpu.make_async_copy(stage, acc.at[dst_v.at[j]], sem).wait()

    @pl.loop(0, 7)
    def _(j):
        pltpu.make_async_copy(stage, acc.at[dst_v.at[j]], sem).wait()

    plsc.subcore_barrier()
    pltpu.sync_copy(acc.at[pl.ds(s * RPS, RPS)],
                    out_hbm.at[c, pl.ds(s * RPS, RPS)])


@functools.lru_cache(maxsize=None)
def _make_prop(f, nb):
    assert CH % nb == 0

    @functools.partial(
        pl.kernel,
        out_type=jax.ShapeDtypeStruct((2, N_PAD, f), jnp.float32),
        mesh=_MESH,
        scratch_types=[
            pltpu.VMEM((CH, K), jnp.int32),
            pltpu.VMEM((CH, K), jnp.int32),
            [pltpu.VMEM((K, f), jnp.float32) for _ in range(nb)],
            pltpu.VMEM_SHARED((N_PAD, f), jnp.float32),
            [pltpu.SemaphoreType.DMA for _ in range(nb)],
            [pltpu.SemaphoreType.DMA for _ in range(nb)],
        ],
        compiler_params=_SC_PARAMS,
    )
    def prop(u_hbm, src_hbm, dst_hbm, out_hbm, src_v, dst_v, st, acc,
             sem_g, sem_s):
        c = lax.axis_index("c")
        s = lax.axis_index("s")
        wid = c * NSC + s
        pltpu.sync_copy(src_hbm.at[wid], src_v)
        pltpu.sync_copy(dst_hbm.at[wid], dst_v)
        _zero_fill(st[0], f)
        _stripe_zero(st[0], acc, s)
        plsc.subcore_barrier()

        for b in range(nb):
            pltpu.async_copy(u_hbm.at[src_v.at[b]], st[b], sem_g[b])

        @pl.loop(0, CH // nb - 1)
        def _(g):
            base = g * nb
            for b in range(nb):
                j = base + b
                pltpu.make_async_copy(u_hbm.at[src_v.at[j]], st[b],
                                      sem_g[b]).wait()
                pltpu.async_copy(st[b], acc.at[dst_v.at[j]], sem_s[b],
                                 add=True)
            for b in range(nb):
                j = base + b
                pltpu.make_async_copy(st[b], acc.at[dst_v.at[j]],
                                      sem_s[b]).wait()
                pltpu.async_copy(u_hbm.at[src_v.at[j + nb]], st[b], sem_g[b])

        base = CH - nb
        for b in range(nb):
            j = base + b
            pltpu.make_async_copy(u_hbm.at[src_v.at[j]], st[b],
                                  sem_g[b]).wait()
            pltpu.async_copy(st[b], acc.at[dst_v.at[j]], sem_s[b], add=True)
        for b in range(nb):
            j = base + b
            pltpu.make_async_copy(st[b], acc.at[dst_v.at[j]], sem_s[b]).wait()

        plsc.subcore_barrier()
        pltpu.sync_copy(acc.at[pl.ds(s * RPS, RPS)],
                        out_hbm.at[c, pl.ds(s * RPS, RPS)])

    return prop


def _row_specs(*widths):
    return [pl.BlockSpec((BR, w), lambda i: (i, 0)) for w in widths]


def _full_spec(shape):
    return pl.BlockSpec(shape, lambda i: (0, 0))


def _tc_pre(p0, p1, x_pad):
    def body(p0_ref, p1_ref, x_ref, dinv_ref, u0_ref):
        deg = p0_ref[...] + p1_ref[...] + 1.0
        dinv = lax.rsqrt(deg)
        dinv_ref[...] = dinv
        u0_ref[...] = x_ref[...] * dinv

    return pl.pallas_call(
        body,
        grid=(N_PAD // BR,),
        in_specs=_row_specs(1, 1, 128),
        out_specs=_row_specs(1, 128),
        out_shape=[jax.ShapeDtypeStruct((N_PAD, 1), jnp.float32),
                   jax.ShapeDtypeStruct((N_PAD, 128), jnp.float32)],
    )(p0, p1, x_pad)


def _tc_layer1(s0, s1, u0, dinv, W1, b1, W2a, W2b):
    def body(s0_ref, s1_ref, u_ref, d_ref, w1_ref, b1_ref, w2a_ref, w2b_ref,
             outa_ref, outb_ref):
        dinv = d_ref[...]
        g1 = dinv * (s0_ref[...] + s1_ref[...] + u_ref[...])
        h1 = jax.nn.relu(
            jnp.dot(g1, w1_ref[...], precision=_PREC,
                    preferred_element_type=jnp.float32) + b1_ref[...])
        outa_ref[...] = dinv * jnp.dot(h1, w2a_ref[...], precision=_PREC,
                                       preferred_element_type=jnp.float32)
        outb_ref[...] = dinv * jnp.dot(h1, w2b_ref[...], precision=_PREC,
                                       preferred_element_type=jnp.float32)

    fa, fb = W2a.shape[1], W2b.shape[1]
    return pl.pallas_call(
        body,
        grid=(N_PAD // BR,),
        in_specs=_row_specs(128, 128, 128, 1)
        + [_full_spec(W1.shape), _full_spec(b1.shape),
           _full_spec(W2a.shape), _full_spec(W2b.shape)],
        out_specs=_row_specs(fa, fb),
        out_shape=[jax.ShapeDtypeStruct((N_PAD, fa), jnp.float32),
                   jax.ShapeDtypeStruct((N_PAD, fb), jnp.float32)],
    )(s0, s1, u0, dinv, W1, b1, W2a, W2b)


def _tc_layer2(s0a, s1a, ua, s0b, s1b, ub, dinv, b2a, b2b, W3a, W3b):
    fa, fb = ua.shape[1], ub.shape[1]
    fo = W3a.shape[1]

    def body(s0a_ref, s1a_ref, ua_ref, s0b_ref, s1b_ref, ub_ref, d_ref,
             ba_ref, bb_ref, wa_ref, wb_ref, out_ref):
        dinv = d_ref[...]
        ga = dinv * (s0a_ref[...] + s1a_ref[...] + ua_ref[...])
        gb = dinv * (s0b_ref[...] + s1b_ref[...] + ub_ref[...])
        ha = jax.nn.relu(ga + ba_ref[...])
        hb = jax.nn.relu(gb + bb_ref[...])
        t = (jnp.dot(ha, wa_ref[...], precision=_PREC,
                     preferred_element_type=jnp.float32)
             + jnp.dot(hb, wb_ref[...], precision=_PREC,
                       preferred_element_type=jnp.float32))
        out_ref[...] = dinv * t

    return pl.pallas_call(
        body,
        grid=(N_PAD // BR,),
        in_specs=_row_specs(fa, fa, fa, fb, fb, fb, 1)
        + [_full_spec(b2a.shape), _full_spec(b2b.shape),
           _full_spec(W3a.shape), _full_spec(W3b.shape)],
        out_specs=_row_specs(fo)[0],
        out_shape=jax.ShapeDtypeStruct((N_PAD, fo), jnp.float32),
    )(s0a, s1a, ua, s0b, s1b, ub, dinv, b2a, b2b, W3a, W3b)


def _tc_layer(s0, s1, u, dinv, bprev, Wnext):
    fi = u.shape[1]
    fo = Wnext.shape[1]

    def body(s0_ref, s1_ref, u_ref, d_ref, b_ref, w_ref, out_ref):
        dinv = d_ref[...]
        g = dinv * (s0_ref[...] + s1_ref[...] + u_ref[...])
        h = jax.nn.relu(g + b_ref[...])
        t = jnp.dot(h, w_ref[...], precision=_PREC,
                    preferred_element_type=jnp.float32)
        out_ref[...] = dinv * t

    return pl.pallas_call(
        body,
        grid=(N_PAD // BR,),
        in_specs=_row_specs(fi, fi, fi, 1)
        + [_full_spec(bprev.shape), _full_spec(Wnext.shape)],
        out_specs=_row_specs(fo)[0],
        out_shape=jax.ShapeDtypeStruct((N_PAD, fo), jnp.float32),
    )(s0, s1, u, dinv, bprev, Wnext)


def _tc_layer4(s0, s1, u, dinv, b4p):
    fi = u.shape[1]

    def body(s0_ref, s1_ref, u_ref, d_ref, b_ref, out_ref):
        dinv = d_ref[...]
        g = dinv * (s0_ref[...] + s1_ref[...] + u_ref[...])
        out_ref[...] = dinv * jax.nn.relu(g + b_ref[...])

    return pl.pallas_call(
        body,
        grid=(N_PAD // BR,),
        in_specs=_row_specs(fi, fi, fi, 1) + [_full_spec(b4p.shape)],
        out_specs=_row_specs(fi)[0],
        out_shape=jax.ShapeDtypeStruct((N_PAD, fi), jnp.float32),
    )(s0, s1, u, dinv, b4p)


def _tc_head(s0, s1, u5, dinv, Wmu, bmu, Wls, bls, eps_pad):
    fi = u5.shape[1]

    def body(s0_ref, s1_ref, u_ref, d_ref, wmu_ref, bmu_ref, wls_ref,
             bls_ref, eps_ref, pz_ref, z_ref):
        g = d_ref[...] * (s0_ref[...] + s1_ref[...] + u_ref[...])
        mu = jnp.dot(g, wmu_ref[...], precision=_PREC,
                     preferred_element_type=jnp.float32) + bmu_ref[...]
        ls = jnp.dot(g, wls_ref[...], precision=_PREC,
                     preferred_element_type=jnp.float32) + bls_ref[...]
        z = mu + eps_ref[...] * jnp.exp(ls)
        m = jnp.max(z, axis=1, keepdims=True)
        pz = z - m - jnp.log(jnp.sum(jnp.exp(z - m), axis=1, keepdims=True))
        pz_ref[...] = pz
        z_ref[...] = z

    return pl.pallas_call(
        body,
        grid=(N_PAD // BR,),
        in_specs=_row_specs(fi, fi, fi, 1)
        + [_full_spec(Wmu.shape), _full_spec(bmu.shape),
           _full_spec(Wls.shape), _full_spec(bls.shape)]
        + _row_specs(21),
        out_specs=_row_specs(21, 21),
        out_shape=[jax.ShapeDtypeStruct((N_PAD, 21), jnp.float32),
                   jax.ShapeDtypeStruct((N_PAD, 21), jnp.float32)],
    )(s0, s1, u5, dinv, Wmu, bmu, Wls, bls, eps_pad)


def _pad2(a, rows, cols):
    return jnp.pad(a, ((0, rows - a.shape[0]), (0, cols - a.shape[1])))


def kernel(x, edge_index, eps, W1, b1, W2, b2, W3, b3, W4, b4,
           W_mu, b_mu, W_ls, b_ls):
    src = edge_index[0]
    dst = edge_index[1]
    pad = E_PAD - N_EDGES
    fill = N_NODES + jnp.arange(pad, dtype=jnp.int32) % (N_PAD - N_NODES)
    src_p = jnp.concatenate([src, fill]).reshape(NW, CH, K)
    dst_p = jnp.concatenate([dst, fill]).reshape(NW, CH, K)

    x_pad = jnp.pad(x, ((0, N_PAD - N_NODES), (0, 0)))
    eps_pad = jnp.pad(eps, ((0, N_PAD - N_NODES), (0, 0)))

    W2a = W2[:, :128]
    W2b = _pad2(W2[:, 128:], 336, 48)
    b2a = b2[:128].reshape(1, 128)
    b2b = jnp.pad(b2[128:], (0, 8)).reshape(1, 48)
    W3a = jnp.pad(W3[:128, :], ((0, 0), (0, 12)))
    W3b = _pad2(W3[128:, :], 48, 96)
    b3p = jnp.pad(b3, (0, 12)).reshape(1, 96)
    W4p = _pad2(W4, 96, 48)
    b4p = jnp.pad(b4, (0, 6)).reshape(1, 48)
    Wmup = _pad2(W_mu, 48, 21)
    Wlsp = _pad2(W_ls, 48, 21)
    b1r = b1.reshape(1, 336)
    bmur = b_mu.reshape(1, 21)
    blsr = b_ls.reshape(1, 21)

    p = _sc_degree(dst_p)
    dinv, u0 = _tc_pre(p[0, :, :1], p[1, :, :1], x_pad)

    s = _make_prop(128, 4)(u0, src_p, dst_p)
    u2a, u2b = _tc_layer1(s[0], s[1], u0, dinv, W1, b1r, W2a, W2b)

    sa = _make_prop(128, 4)(u2a, src_p, dst_p)
    sb = _make_prop(48, 8)(u2b, src_p, dst_p)
    u3 = _tc_layer2(sa[0], sa[1], u2a, sb[0], sb[1], u2b, dinv,
                    b2a, b2b, W3a, W3b)

    s = _make_prop(96, 6)(u3, src_p, dst_p)
    u4 = _tc_layer(s[0], s[1], u3, dinv, b3p, W4p)

    s = _make_prop(48, 8)(u4, src_p, dst_p)
    u5 = _tc_layer4(s[0], s[1], u4, dinv, b4p)

    s = _make_prop(48, 8)(u5, src_p, dst_p)
    pz, z = _tc_head(s[0], s[1], u5, dinv, Wmup, bmur, Wlsp, blsr, eps_pad)

    return (pz[:N_NODES], z[:N_NODES])


# speedup vs baseline: 29.8595x; 1.0538x over previous
import functools

import jax
import jax.numpy as jnp
from jax import lax
from jax.experimental import pallas as pl
from jax.experimental.pallas import tpu as pltpu
from jax.experimental.pallas import tpu_sc as plsc

N_NODES = 10000
N_EDGES = 320000
N_PAD = 10240
NW = 32
NSC = 16
K = 48
CH = 216
E_PAD = NW * CH * K
RPS = N_PAD // NSC
BR = 1280

_MESH = plsc.VectorSubcoreMesh(core_axis_name="c", subcore_axis_name="s")
_PREC = jax.lax.Precision.DEFAULT
_SC_PARAMS = pltpu.CompilerParams(use_tc_tiling_on_sc=False)


def _zero_fill(stage, f):
    zeros = jnp.zeros((16,), jnp.float32)

    @pl.loop(0, K)
    def _(r):
        for cc in range(f // 16):
            stage[r, pl.ds(cc * 16, 16)] = zeros


def _stripe_zero(stage, acc, s):
    for b in range(RPS // K):
        pltpu.sync_copy(stage, acc.at[pl.ds(s * RPS + b * K, K)])
    rem = RPS % K
    if rem:
        pltpu.sync_copy(stage.at[pl.ds(0, rem)],
                        acc.at[pl.ds(s * RPS + (RPS // K) * K, rem)])


@functools.partial(
    pl.kernel,
    out_type=jax.ShapeDtypeStruct((2, N_PAD, 16), jnp.float32),
    mesh=_MESH,
    scratch_types=[
        pltpu.VMEM((CH, K), jnp.int32),
        pltpu.VMEM((K, 16), jnp.float32),
        pltpu.VMEM_SHARED((N_PAD, 16), jnp.float32),
        pltpu.SemaphoreType.DMA,
    ],
    compiler_params=_SC_PARAMS,
)
def _sc_degree(dst_hbm, out_hbm, dst_v, stage, acc, sem):
    c = lax.axis_index("c")
    s = lax.axis_index("s")
    wid = c * NSC + s
    pltpu.sync_copy(dst_hbm.at[wid], dst_v)
    _zero_fill(stage, 16)
    _stripe_zero(stage, acc, s)
    basis = jnp.where(lax.iota(jnp.int32, 16) == 0, 1.0, 0.0)

    @pl.loop(0, K)
    def _(r):
        stage[r, pl.ds(0, 16)] = basis

    plsc.subcore_barrier()

    @pl.loop(0, CH)
    def _(j):
        pltpu.async_copy(stage, acc.at[dst_v.at[j]], sem, add=True)

        @pl.when(j >= 7)
        def _():
            pltpu.make_async_copy(stage, acc.at[dst_v.at[j]], sem).wait()

    @pl.loop(0, 7)
    def _(j):
        pltpu.make_async_copy(stage, acc.at[dst_v.at[j]], sem).wait()

    plsc.subcore_barrier()
    pltpu.sync_copy(acc.at[pl.ds(s * RPS, RPS)],
                    out_hbm.at[c, pl.ds(s * RPS, RPS)])


@functools.lru_cache(maxsize=None)
def _make_prop(f, nb):
    assert CH % nb == 0

    @functools.partial(
        pl.kernel,
        out_type=jax.ShapeDtypeStruct((2, N_PAD, f), jnp.float32),
        mesh=_MESH,
        scratch_types=[
            pltpu.VMEM((CH, K), jnp.int32),
            pltpu.VMEM((CH, K), jnp.int32),
            [pltpu.VMEM((K, f), jnp.float32) for _ in range(nb)],
            pltpu.VMEM_SHARED((N_PAD, f), jnp.float32),
            [pltpu.SemaphoreType.DMA for _ in range(nb)],
            [pltpu.SemaphoreType.DMA for _ in range(nb)],
        ],
        compiler_params=_SC_PARAMS,
    )
    def prop(u_hbm, src_hbm, dst_hbm, out_hbm, src_v, dst_v, st, acc,
             sem_g, sem_s):
        c = lax.axis_index("c")
        s = lax.axis_index("s")
        wid = c * NSC + s
        pltpu.sync_copy(src_hbm.at[wid], src_v)
        pltpu.sync_copy(dst_hbm.at[wid], dst_v)
        _zero_fill(st[0], f)
        _stripe_zero(st[0], acc, s)
        plsc.subcore_barrier()

        for b in range(nb):
            pltpu.async_copy(u_hbm.at[src_v.at[b]], st[b], sem_g[b])

        @pl.loop(0, CH // nb - 1)
        def _(g):
            base = g * nb
            for b in range(nb):
                j = base + b
                pltpu.make_async_copy(u_hbm.at[src_v.at[j]], st[b],
                                      sem_g[b]).wait()
                pltpu.async_copy(st[b], acc.at[dst_v.at[j]], sem_s[b],
                                 add=True)
            for b in range(nb):
                j = base + b
                pltpu.make_async_copy(st[b], acc.at[dst_v.at[j]],
                                      sem_s[b]).wait()
                pltpu.async_copy(u_hbm.at[src_v.at[j + nb]], st[b], sem_g[b])

        base = CH - nb
        for b in range(nb):
            j = base + b
            pltpu.make_async_copy(u_hbm.at[src_v.at[j]], st[b],
                                  sem_g[b]).wait()
            pltpu.async_copy(st[b], acc.at[dst_v.at[j]], sem_s[b], add=True)
        for b in range(nb):
            j = base + b
            pltpu.make_async_copy(st[b], acc.at[dst_v.at[j]], sem_s[b]).wait()

        plsc.subcore_barrier()
        pltpu.sync_copy(acc.at[pl.ds(s * RPS, RPS)],
                        out_hbm.at[c, pl.ds(s * RPS, RPS)])

    return prop


def _row_specs(*widths):
    return [pl.BlockSpec((BR, w), lambda i: (i, 0)) for w in widths]


def _full_spec(shape):
    return pl.BlockSpec(shape, lambda i: (0, 0))


def _tc_pre(p0, p1, x_pad):
    def body(p0_ref, p1_ref, x_ref, dinv_ref, u0_ref):
        deg = p0_ref[...] + p1_ref[...] + 1.0
        dinv = lax.rsqrt(deg)
        dinv_ref[...] = dinv
        u0_ref[...] = x_ref[...] * dinv

    return pl.pallas_call(
        body,
        grid=(N_PAD // BR,),
        in_specs=_row_specs(1, 1, 128),
        out_specs=_row_specs(1, 128),
        out_shape=[jax.ShapeDtypeStruct((N_PAD, 1), jnp.float32),
                   jax.ShapeDtypeStruct((N_PAD, 128), jnp.float32)],
    )(p0, p1, x_pad)


def _tc_layer1(s0, s1, u0, dinv, W1, b1, W2a, W2b):
    def body(s0_ref, s1_ref, u_ref, d_ref, w1_ref, b1_ref, w2a_ref, w2b_ref,
             outa_ref, outb_ref):
        dinv = d_ref[...]
        g1 = dinv * (s0_ref[...] + s1_ref[...] + u_ref[...])
        h1 = jax.nn.relu(
            jnp.dot(g1, w1_ref[...], precision=_PREC,
                    preferred_element_type=jnp.float32) + b1_ref[...])
        outa_ref[...] = dinv * jnp.dot(h1, w2a_ref[...], precision=_PREC,
                                       preferred_element_type=jnp.float32)
        outb_ref[...] = dinv * jnp.dot(h1, w2b_ref[...], precision=_PREC,
                                       preferred_element_type=jnp.float32)

    fa, fb = W2a.shape[1], W2b.shape[1]
    return pl.pallas_call(
        body,
        grid=(N_PAD // BR,),
        in_specs=_row_specs(128, 128, 128, 1)
        + [_full_spec(W1.shape), _full_spec(b1.shape),
           _full_spec(W2a.shape), _full_spec(W2b.shape)],
        out_specs=_row_specs(fa, fb),
        out_shape=[jax.ShapeDtypeStruct((N_PAD, fa), jnp.float32),
                   jax.ShapeDtypeStruct((N_PAD, fb), jnp.float32)],
    )(s0, s1, u0, dinv, W1, b1, W2a, W2b)


def _tc_layer2(s0a, s1a, ua, s0b, s1b, ub, dinv, b2a, b2b, W3a, W3b):
    fa, fb = ua.shape[1], ub.shape[1]
    fo = W3a.shape[1]

    def body(s0a_ref, s1a_ref, ua_ref, s0b_ref, s1b_ref, ub_ref, d_ref,
             ba_ref, bb_ref, wa_ref, wb_ref, out_ref):
        dinv = d_ref[...]
        ga = dinv * (s0a_ref[...] + s1a_ref[...] + ua_ref[...])
        gb = dinv * (s0b_ref[...] + s1b_ref[...] + ub_ref[...])
        ha = jax.nn.relu(ga + ba_ref[...])
        hb = jax.nn.relu(gb + bb_ref[...])
        t = (jnp.dot(ha, wa_ref[...], precision=_PREC,
                     preferred_element_type=jnp.float32)
             + jnp.dot(hb, wb_ref[...], precision=_PREC,
                       preferred_element_type=jnp.float32))
        out_ref[...] = dinv * t

    return pl.pallas_call(
        body,
        grid=(N_PAD // BR,),
        in_specs=_row_specs(fa, fa, fa, fb, fb, fb, 1)
        + [_full_spec(b2a.shape), _full_spec(b2b.shape),
           _full_spec(W3a.shape), _full_spec(W3b.shape)],
        out_specs=_row_specs(fo)[0],
        out_shape=jax.ShapeDtypeStruct((N_PAD, fo), jnp.float32),
    )(s0a, s1a, ua, s0b, s1b, ub, dinv, b2a, b2b, W3a, W3b)


def _tc_layer(s0, s1, u, dinv, bprev, Wnext):
    fi = u.shape[1]
    fo = Wnext.shape[1]

    def body(s0_ref, s1_ref, u_ref, d_ref, b_ref, w_ref, out_ref):
        dinv = d_ref[...]
        g = dinv * (s0_ref[...] + s1_ref[...] + u_ref[...])
        h = jax.nn.relu(g + b_ref[...])
        t = jnp.dot(h, w_ref[...], precision=_PREC,
                    preferred_element_type=jnp.float32)
        out_ref[...] = dinv * t

    return pl.pallas_call(
        body,
        grid=(N_PAD // BR,),
        in_specs=_row_specs(fi, fi, fi, 1)
        + [_full_spec(bprev.shape), _full_spec(Wnext.shape)],
        out_specs=_row_specs(fo)[0],
        out_shape=jax.ShapeDtypeStruct((N_PAD, fo), jnp.float32),
    )(s0, s1, u, dinv, bprev, Wnext)


def _tc_layer4(s0, s1, u, dinv, b4p):
    fi = u.shape[1]

    def body(s0_ref, s1_ref, u_ref, d_ref, b_ref, out_ref):
        dinv = d_ref[...]
        g = dinv * (s0_ref[...] + s1_ref[...] + u_ref[...])
        out_ref[...] = dinv * jax.nn.relu(g + b_ref[...])

    return pl.pallas_call(
        body,
        grid=(N_PAD // BR,),
        in_specs=_row_specs(fi, fi, fi, 1) + [_full_spec(b4p.shape)],
        out_specs=_row_specs(fi)[0],
        out_shape=jax.ShapeDtypeStruct((N_PAD, fi), jnp.float32),
    )(s0, s1, u, dinv, b4p)


def _tc_head(s0, s1, u5, dinv, Wmu, bmu, Wls, bls, eps_pad):
    fi = u5.shape[1]

    def body(s0_ref, s1_ref, u_ref, d_ref, wmu_ref, bmu_ref, wls_ref,
             bls_ref, eps_ref, pz_ref, z_ref):
        g = d_ref[...] * (s0_ref[...] + s1_ref[...] + u_ref[...])
        mu = jnp.dot(g, wmu_ref[...], precision=_PREC,
                     preferred_element_type=jnp.float32) + bmu_ref[...]
        ls = jnp.dot(g, wls_ref[...], precision=_PREC,
                     preferred_element_type=jnp.float32) + bls_ref[...]
        z = mu + eps_ref[...] * jnp.exp(ls)
        m = jnp.max(z, axis=1, keepdims=True)
        pz = z - m - jnp.log(jnp.sum(jnp.exp(z - m), axis=1, keepdims=True))
        pz_ref[...] = pz
        z_ref[...] = z

    return pl.pallas_call(
        body,
        grid=(N_PAD // BR,),
        in_specs=_row_specs(fi, fi, fi, 1)
        + [_full_spec(Wmu.shape), _full_spec(bmu.shape),
           _full_spec(Wls.shape), _full_spec(bls.shape)]
        + _row_specs(21),
        out_specs=_row_specs(21, 21),
        out_shape=[jax.ShapeDtypeStruct((N_PAD, 21), jnp.float32),
                   jax.ShapeDtypeStruct((N_PAD, 21), jnp.float32)],
    )(s0, s1, u5, dinv, Wmu, bmu, Wls, bls, eps_pad)


def _pad2(a, rows, cols):
    return jnp.pad(a, ((0, rows - a.shape[0]), (0, cols - a.shape[1])))


def kernel(x, edge_index, eps, W1, b1, W2, b2, W3, b3, W4, b4,
           W_mu, b_mu, W_ls, b_ls):
    src = edge_index[0]
    dst = edge_index[1]
    pad = E_PAD - N_EDGES
    fill = N_NODES + jnp.arange(pad, dtype=jnp.int32) % (N_PAD - N_NODES)
    src_p = jnp.concatenate([src, fill]).reshape(NW, CH, K)
    dst_p = jnp.concatenate([dst, fill]).reshape(NW, CH, K)

    x_pad = jnp.pad(x, ((0, N_PAD - N_NODES), (0, 0)))
    eps_pad = jnp.pad(eps, ((0, N_PAD - N_NODES), (0, 0)))

    W2a = W2[:, :128]
    W2b = _pad2(W2[:, 128:], 336, 48)
    b2a = b2[:128].reshape(1, 128)
    b2b = jnp.pad(b2[128:], (0, 8)).reshape(1, 48)
    W3a = jnp.pad(W3[:128, :], ((0, 0), (0, 12)))
    W3b = _pad2(W3[128:, :], 48, 96)
    b3p = jnp.pad(b3, (0, 12)).reshape(1, 96)
    W4p = _pad2(W4, 96, 48)
    b4p = jnp.pad(b4, (0, 6)).reshape(1, 48)
    Wmup = _pad2(W_mu, 48, 21)
    Wlsp = _pad2(W_ls, 48, 21)
    b1r = b1.reshape(1, 336)
    bmur = b_mu.reshape(1, 21)
    blsr = b_ls.reshape(1, 21)

    p = _sc_degree(dst_p)
    dinv, u0 = _tc_pre(p[0, :, :1], p[1, :, :1], x_pad)

    s = _make_prop(128, 4)(u0, src_p, dst_p)
    u2a, u2b = _tc_layer1(s[0], s[1], u0, dinv, W1, b1r, W2a, W2b)

    sa = _make_prop(128, 4)(u2a, src_p, dst_p)
    sb = _make_prop(48, 12)(u2b, src_p, dst_p)
    u3 = _tc_layer2(sa[0], sa[1], u2a, sb[0], sb[1], u2b, dinv,
                    b2a, b2b, W3a, W3b)

    s = _make_prop(96, 8)(u3, src_p, dst_p)
    u4 = _tc_layer(s[0], s[1], u3, dinv, b3p, W4p)

    s = _make_prop(48, 12)(u4, src_p, dst_p)
    u5 = _tc_layer4(s[0], s[1], u4, dinv, b4p)

    s = _make_prop(48, 12)(u5, src_p, dst_p)
    pz, z = _tc_head(s[0], s[1], u5, dinv, Wmup, bmur, Wlsp, blsr, eps_pad)

    return (pz[:N_NODES], z[:N_NODES])


# speedup vs baseline: 30.1701x; 1.0104x over previous
import functools

import jax
import jax.numpy as jnp
from jax import lax
from jax.experimental import pallas as pl
from jax.experimental.pallas import tpu as pltpu
from jax.experimental.pallas import tpu_sc as plsc

N_NODES = 10000
N_EDGES = 320000
N_PAD = 10240
NW = 32
NSC = 16
K = 48
CH = 216
E_PAD = NW * CH * K
RPS = N_PAD // NSC
BR = 2560

_MESH = plsc.VectorSubcoreMesh(core_axis_name="c", subcore_axis_name="s")
_PREC = jax.lax.Precision.DEFAULT
_SC_PARAMS = pltpu.CompilerParams(use_tc_tiling_on_sc=False)


def _zero_fill(stage, f):
    zeros = jnp.zeros((16,), jnp.float32)

    @pl.loop(0, K)
    def _(r):
        for cc in range(f // 16):
            stage[r, pl.ds(cc * 16, 16)] = zeros


def _stripe_zero(stage, acc, s):
    for b in range(RPS // K):
        pltpu.sync_copy(stage, acc.at[pl.ds(s * RPS + b * K, K)])
    rem = RPS % K
    if rem:
        pltpu.sync_copy(stage.at[pl.ds(0, rem)],
                        acc.at[pl.ds(s * RPS + (RPS // K) * K, rem)])


@functools.partial(
    pl.kernel,
    out_type=jax.ShapeDtypeStruct((2, N_PAD, 16), jnp.float32),
    mesh=_MESH,
    scratch_types=[
        pltpu.VMEM((CH, K), jnp.int32),
        pltpu.VMEM((K, 16), jnp.float32),
        pltpu.VMEM_SHARED((N_PAD, 16), jnp.float32),
        pltpu.SemaphoreType.DMA,
    ],
    compiler_params=_SC_PARAMS,
)
def _sc_degree(dst_hbm, out_hbm, dst_v, stage, acc, sem):
    c = lax.axis_index("c")
    s = lax.axis_index("s")
    wid = c * NSC + s
    pltpu.sync_copy(dst_hbm.at[wid], dst_v)
    _zero_fill(stage, 16)
    _stripe_zero(stage, acc, s)
    basis = jnp.where(lax.iota(jnp.int32, 16) == 0, 1.0, 0.0)

    @pl.loop(0, K)
    def _(r):
        stage[r, pl.ds(0, 16)] = basis

    plsc.subcore_barrier()

    @pl.loop(0, CH)
    def _(j):
        pltpu.async_copy(stage, acc.at[dst_v.at[j]], sem, add=True)

        @pl.when(j >= 15)
        def _():
            pltpu.make_async_copy(stage, acc.at[dst_v.at[j]], sem).wait()

    @pl.loop(0, 15)
    def _(j):
        pltpu.make_async_copy(stage, acc.at[dst_v.at[j]], sem).wait()

    plsc.subcore_barrier()
    pltpu.sync_copy(acc.at[pl.ds(s * RPS, RPS)],
                    out_hbm.at[c, pl.ds(s * RPS, RPS)])


@functools.lru_cache(maxsize=None)
def _make_prop(f, nb):
    assert CH % nb == 0

    @functools.partial(
        pl.kernel,
        out_type=jax.ShapeDtypeStruct((2, N_PAD, f), jnp.float32),
        mesh=_MESH,
        scratch_types=[
            pltpu.VMEM((CH, K), jnp.int32),
            pltpu.VMEM((CH, K), jnp.int32),
            [pltpu.VMEM((K, f), jnp.float32) for _ in range(nb)],
            pltpu.VMEM_SHARED((N_PAD, f), jnp.float32),
            [pltpu.SemaphoreType.DMA for _ in range(nb)],
            [pltpu.SemaphoreType.DMA for _ in range(nb)],
        ],
        compiler_params=_SC_PARAMS,
    )
    def prop(u_hbm, src_hbm, dst_hbm, out_hbm, src_v, dst_v, st, acc,
             sem_g, sem_s):
        c = lax.axis_index("c")
        s = lax.axis_index("s")
        wid = c * NSC + s
        pltpu.sync_copy(src_hbm.at[wid], src_v)
        pltpu.sync_copy(dst_hbm.at[wid], dst_v)
        _zero_fill(st[0], f)
        _stripe_zero(st[0], acc, s)
        plsc.subcore_barrier()

        for b in range(nb):
            pltpu.async_copy(u_hbm.at[src_v.at[b]], st[b], sem_g[b])

        @pl.loop(0, CH // nb - 1)
        def _(g):
            base = g * nb
            for b in range(nb):
                j = base + b
                pltpu.make_async_copy(u_hbm.at[src_v.at[j]], st[b],
                                      sem_g[b]).wait()
                pltpu.async_copy(st[b], acc.at[dst_v.at[j]], sem_s[b],
                                 add=True)
            for b in range(nb):
                j = base + b
                pltpu.make_async_copy(st[b], acc.at[dst_v.at[j]],
                                      sem_s[b]).wait()
                pltpu.async_copy(u_hbm.at[src_v.at[j + nb]], st[b], sem_g[b])

        base = CH - nb
        for b in range(nb):
            j = base + b
            pltpu.make_async_copy(u_hbm.at[src_v.at[j]], st[b],
                                  sem_g[b]).wait()
            pltpu.async_copy(st[b], acc.at[dst_v.at[j]], sem_s[b], add=True)
        for b in range(nb):
            j = base + b
            pltpu.make_async_copy(st[b], acc.at[dst_v.at[j]], sem_s[b]).wait()

        plsc.subcore_barrier()
        pltpu.sync_copy(acc.at[pl.ds(s * RPS, RPS)],
                        out_hbm.at[c, pl.ds(s * RPS, RPS)])

    return prop


def _row_specs(*widths):
    return [pl.BlockSpec((BR, w), lambda i: (i, 0)) for w in widths]


def _full_spec(shape):
    return pl.BlockSpec(shape, lambda i: (0, 0))


def _tc_pre(p0, p1, x_pad):
    def body(p0_ref, p1_ref, x_ref, dinv_ref, u0_ref):
        deg = p0_ref[...] + p1_ref[...] + 1.0
        dinv = lax.rsqrt(deg)
        dinv_ref[...] = dinv
        u0_ref[...] = x_ref[...] * dinv

    return pl.pallas_call(
        body,
        grid=(N_PAD // BR,),
        in_specs=_row_specs(1, 1, 128),
        out_specs=_row_specs(1, 128),
        out_shape=[jax.ShapeDtypeStruct((N_PAD, 1), jnp.float32),
                   jax.ShapeDtypeStruct((N_PAD, 128), jnp.float32)],
    )(p0, p1, x_pad)


def _tc_layer1(s0, s1, u0, dinv, W1, b1, W2a, W2b):
    def body(s0_ref, s1_ref, u_ref, d_ref, w1_ref, b1_ref, w2a_ref, w2b_ref,
             outa_ref, outb_ref):
        dinv = d_ref[...]
        g1 = dinv * (s0_ref[...] + s1_ref[...] + u_ref[...])
        h1 = jax.nn.relu(
            jnp.dot(g1, w1_ref[...], precision=_PREC,
                    preferred_element_type=jnp.float32) + b1_ref[...])
        outa_ref[...] = dinv * jnp.dot(h1, w2a_ref[...], precision=_PREC,
                                       preferred_element_type=jnp.float32)
        outb_ref[...] = dinv * jnp.dot(h1, w2b_ref[...], precision=_PREC,
                                       preferred_element_type=jnp.float32)

    fa, fb = W2a.shape[1], W2b.shape[1]
    return pl.pallas_call(
        body,
        grid=(N_PAD // BR,),
        in_specs=_row_specs(128, 128, 128, 1)
        + [_full_spec(W1.shape), _full_spec(b1.shape),
           _full_spec(W2a.shape), _full_spec(W2b.shape)],
        out_specs=_row_specs(fa, fb),
        out_shape=[jax.ShapeDtypeStruct((N_PAD, fa), jnp.float32),
                   jax.ShapeDtypeStruct((N_PAD, fb), jnp.float32)],
    )(s0, s1, u0, dinv, W1, b1, W2a, W2b)


def _tc_layer2(s0a, s1a, ua, s0b, s1b, ub, dinv, b2a, b2b, W3a, W3b):
    fa, fb = ua.shape[1], ub.shape[1]
    fo = W3a.shape[1]

    def body(s0a_ref, s1a_ref, ua_ref, s0b_ref, s1b_ref, ub_ref, d_ref,
             ba_ref, bb_ref, wa_ref, wb_ref, out_ref):
        dinv = d_ref[...]
        ga = dinv * (s0a_ref[...] + s1a_ref[...] + ua_ref[...])
        gb = dinv * (s0b_ref[...] + s1b_ref[...] + ub_ref[...])
        ha = jax.nn.relu(ga + ba_ref[...])
        hb = jax.nn.relu(gb + bb_ref[...])
        t = (jnp.dot(ha, wa_ref[...], precision=_PREC,
                     preferred_element_type=jnp.float32)
             + jnp.dot(hb, wb_ref[...], precision=_PREC,
                       preferred_element_type=jnp.float32))
        out_ref[...] = dinv * t

    return pl.pallas_call(
        body,
        grid=(N_PAD // BR,),
        in_specs=_row_specs(fa, fa, fa, fb, fb, fb, 1)
        + [_full_spec(b2a.shape), _full_spec(b2b.shape),
           _full_spec(W3a.shape), _full_spec(W3b.shape)],
        out_specs=_row_specs(fo)[0],
        out_shape=jax.ShapeDtypeStruct((N_PAD, fo), jnp.float32),
    )(s0a, s1a, ua, s0b, s1b, ub, dinv, b2a, b2b, W3a, W3b)


def _tc_layer(s0, s1, u, dinv, bprev, Wnext):
    fi = u.shape[1]
    fo = Wnext.shape[1]

    def body(s0_ref, s1_ref, u_ref, d_ref, b_ref, w_ref, out_ref):
        dinv = d_ref[...]
        g = dinv * (s0_ref[...] + s1_ref[...] + u_ref[...])
        h = jax.nn.relu(g + b_ref[...])
        t = jnp.dot(h, w_ref[...], precision=_PREC,
                    preferred_element_type=jnp.float32)
        out_ref[...] = dinv * t

    return pl.pallas_call(
        body,
        grid=(N_PAD // BR,),
        in_specs=_row_specs(fi, fi, fi, 1)
        + [_full_spec(bprev.shape), _full_spec(Wnext.shape)],
        out_specs=_row_specs(fo)[0],
        out_shape=jax.ShapeDtypeStruct((N_PAD, fo), jnp.float32),
    )(s0, s1, u, dinv, bprev, Wnext)


def _tc_layer4(s0, s1, u, dinv, b4p):
    fi = u.shape[1]

    def body(s0_ref, s1_ref, u_ref, d_ref, b_ref, out_ref):
        dinv = d_ref[...]
        g = dinv * (s0_ref[...] + s1_ref[...] + u_ref[...])
        out_ref[...] = dinv * jax.nn.relu(g + b_ref[...])

    return pl.pallas_call(
        body,
        grid=(N_PAD // BR,),
        in_specs=_row_specs(fi, fi, fi, 1) + [_full_spec(b4p.shape)],
        out_specs=_row_specs(fi)[0],
        out_shape=jax.ShapeDtypeStruct((N_PAD, fi), jnp.float32),
    )(s0, s1, u, dinv, b4p)


def _tc_head(s0, s1, u5, dinv, Wmu, bmu, Wls, bls, eps_pad):
    fi = u5.shape[1]

    def body(s0_ref, s1_ref, u_ref, d_ref, wmu_ref, bmu_ref, wls_ref,
             bls_ref, eps_ref, pz_ref, z_ref):
        g = d_ref[...] * (s0_ref[...] + s1_ref[...] + u_ref[...])
        mu = jnp.dot(g, wmu_ref[...], precision=_PREC,
                     preferred_element_type=jnp.float32) + bmu_ref[...]
        ls = jnp.dot(g, wls_ref[...], precision=_PREC,
                     preferred_element_type=jnp.float32) + bls_ref[...]
        z = mu + eps_ref[...] * jnp.exp(ls)
        m = jnp.max(z, axis=1, keepdims=True)
        pz = z - m - jnp.log(jnp.sum(jnp.exp(z - m), axis=1, keepdims=True))
        pz_ref[...] = pz
        z_ref[...] = z

    return pl.pallas_call(
        body,
        grid=(N_PAD // BR,),
        in_specs=_row_specs(fi, fi, fi, 1)
        + [_full_spec(Wmu.shape), _full_spec(bmu.shape),
           _full_spec(Wls.shape), _full_spec(bls.shape)]
        + _row_specs(21),
        out_specs=_row_specs(21, 21),
        out_shape=[jax.ShapeDtypeStruct((N_PAD, 21), jnp.float32),
                   jax.ShapeDtypeStruct((N_PAD, 21), jnp.float32)],
    )(s0, s1, u5, dinv, Wmu, bmu, Wls, bls, eps_pad)


def _pad2(a, rows, cols):
    return jnp.pad(a, ((0, rows - a.shape[0]), (0, cols - a.shape[1])))


def kernel(x, edge_index, eps, W1, b1, W2, b2, W3, b3, W4, b4,
           W_mu, b_mu, W_ls, b_ls):
    src = edge_index[0]
    dst = edge_index[1]
    pad = E_PAD - N_EDGES
    fill = N_NODES + jnp.arange(pad, dtype=jnp.int32) % (N_PAD - N_NODES)
    src_p = jnp.concatenate([src, fill]).reshape(NW, CH, K)
    dst_p = jnp.concatenate([dst, fill]).reshape(NW, CH, K)

    x_pad = jnp.pad(x, ((0, N_PAD - N_NODES), (0, 0)))
    eps_pad = jnp.pad(eps, ((0, N_PAD - N_NODES), (0, 0)))

    W2a = W2[:, :128]
    W2b = _pad2(W2[:, 128:], 336, 48)
    b2a = b2[:128].reshape(1, 128)
    b2b = jnp.pad(b2[128:], (0, 8)).reshape(1, 48)
    W3a = jnp.pad(W3[:128, :], ((0, 0), (0, 12)))
    W3b = _pad2(W3[128:, :], 48, 96)
    b3p = jnp.pad(b3, (0, 12)).reshape(1, 96)
    W4p = _pad2(W4, 96, 48)
    b4p = jnp.pad(b4, (0, 6)).reshape(1, 48)
    Wmup = _pad2(W_mu, 48, 21)
    Wlsp = _pad2(W_ls, 48, 21)
    b1r = b1.reshape(1, 336)
    bmur = b_mu.reshape(1, 21)
    blsr = b_ls.reshape(1, 21)

    p = _sc_degree(dst_p)
    dinv, u0 = _tc_pre(p[0, :, :1], p[1, :, :1], x_pad)

    s = _make_prop(128, 4)(u0, src_p, dst_p)
    u2a, u2b = _tc_layer1(s[0], s[1], u0, dinv, W1, b1r, W2a, W2b)

    sa = _make_prop(128, 4)(u2a, src_p, dst_p)
    sb = _make_prop(48, 12)(u2b, src_p, dst_p)
    u3 = _tc_layer2(sa[0], sa[1], u2a, sb[0], sb[1], u2b, dinv,
                    b2a, b2b, W3a, W3b)

    s = _make_prop(96, 9)(u3, src_p, dst_p)
    u4 = _tc_layer(s[0], s[1], u3, dinv, b3p, W4p)

    s = _make_prop(48, 12)(u4, src_p, dst_p)
    u5 = _tc_layer4(s[0], s[1], u4, dinv, b4p)

    s = _make_prop(48, 12)(u5, src_p, dst_p)
    pz, z = _tc_head(s[0], s[1], u5, dinv, Wmup, bmur, Wlsp, blsr, eps_pad)

    return (pz[:N_NODES], z[:N_NODES])
